# baseline (speedup 1.0000x reference)
import functools

import jax
import jax.numpy as jnp
from jax import lax
from jax.experimental import pallas as pl
from jax.experimental.pallas import tpu as pltpu
from jax.experimental.pallas import tpu_sc as plsc

L_TX, L_SP, L_TP = 16, 8, 8
D = 768
B = 4096
TOK = L_TX + L_SP + L_TP
ROWS = B * TOK

NC, NS = 2, 16
NW = NC * NS
SC_ROWS = 65536
TC_ROWS = ROWS - SC_ROWS
ROWS_PER_W = SC_ROWS // NW
CHUNK = 64
NCHUNK = ROWS_PER_W // CHUNK
LANES = 16
DSTEPS = D // LANES
TC_BLK = 512
TC_NB = TC_ROWS // TC_BLK


@functools.partial(
    pl.kernel,
    out_type=jax.ShapeDtypeStruct((ROWS * D,), jnp.float32),
    mesh=plsc.VectorSubcoreMesh(core_axis_name="c", subcore_axis_name="s"),
    scratch_types=[
        pltpu.VMEM((ROWS_PER_W,), jnp.int32),
        pltpu.VMEM((TOK * D,), jnp.float32),
        pltpu.VMEM((CHUNK * D,), jnp.float32),
        pltpu.VMEM((CHUNK * D,), jnp.float32),
        pltpu.SemaphoreType.DMA,
        pltpu.SemaphoreType.DMA,
    ],
)
def _sc_gather(table_hbm, idx_hbm, out_hbm, idx_v, tab_v, buf0, buf1,
               ss0, ss1):
    wid = lax.axis_index("s") * NC + lax.axis_index("c")
    base = wid * ROWS_PER_W
    pltpu.sync_copy(table_hbm, tab_v)
    pltpu.sync_copy(idx_hbm.at[pl.ds(base, ROWS_PER_W)], idx_v)

    bufs = (buf0, buf1)
    ssem = (ss0, ss1)

    LAG = 8

    def build(b, c):
        def group(g, carry):
            srcs = idx_v[pl.ds(c * CHUNK + g * LANES, LANES)] * D
            dst0 = g * LANES * D
            for j in range(LANES):
                src = srcs[j]
                dst = dst0 + j * D
                vals = [None] * DSTEPS
                for d in range(DSTEPS):
                    vals[d] = tab_v[pl.ds(src + d * LANES, LANES)]
                    if d >= LAG:
                        bufs[b][pl.ds(dst + (d - LAG) * LANES, LANES)] = (
                            vals[d - LAG])
                for d in range(DSTEPS - LAG, DSTEPS):
                    bufs[b][pl.ds(dst + d * LANES, LANES)] = vals[d]
            return carry
        lax.fori_loop(0, CHUNK // LANES, group, 0)

    def start_scatter(b, c):
        pltpu.async_copy(
            bufs[b], out_hbm.at[pl.ds((base + c * CHUNK) * D, CHUNK * D)],
            ssem[b])

    def wait_scatter(b):
        pltpu.make_async_copy(
            bufs[b], out_hbm.at[pl.ds(base * D, CHUNK * D)], ssem[b]).wait()

    def body(i, carry):
        c0 = i * 2

        @pl.when(i > 0)
        def _():
            wait_scatter(0)
        build(0, c0)
        start_scatter(0, c0)

        @pl.when(i > 0)
        def _():
            wait_scatter(1)
        build(1, c0 + 1)
        start_scatter(1, c0 + 1)
        return carry

    lax.fori_loop(0, NCHUNK // 2, body, 0)
    wait_scatter(0)
    wait_scatter(1)


def _tc_body(full_ref, idx_ref, tab_ref, out_ref):
    idxv = idx_ref[0, 0, :]
    onehot = (idxv[:, None]
              == lax.broadcasted_iota(jnp.int32, (TC_BLK, TOK), 1))
    out_ref[...] = jnp.dot(onehot.astype(jnp.float32), tab_ref[...],
                           preferred_element_type=jnp.float32)


_tc_fill = pl.pallas_call(
    _tc_body,
    grid=(TC_NB,),
    in_specs=[
        pl.BlockSpec(memory_space=pl.ANY),
        pl.BlockSpec((1, 1, TC_BLK), lambda i: (i, 0, 0)),
        pl.BlockSpec((TOK, D), lambda i: (0, 0)),
    ],
    out_specs=pl.BlockSpec((TC_BLK, D), lambda i: (SC_ROWS // TC_BLK + i, 0)),
    out_shape=jax.ShapeDtypeStruct((ROWS, D), jnp.float32),
    input_output_aliases={0: 0},
    compiler_params=pltpu.CompilerParams(
        dimension_semantics=("arbitrary",)),
)


def kernel(P_gn_txt, P_gn_ViT, P_gn_temp, idx_txt, idx_vit, idx_temp):
    table = jnp.concatenate([P_gn_txt, P_gn_ViT, P_gn_temp], axis=0)
    idx = jnp.concatenate(
        [idx_txt, idx_vit + L_TX, idx_temp + (L_TX + L_SP)], axis=1
    ).reshape(ROWS)
    sc_part = _sc_gather(table.reshape(TOK * D), idx)
    idx_tc = idx[SC_ROWS:].reshape(TC_NB, 1, TC_BLK)
    out = _tc_fill(sc_part.reshape(ROWS, D), idx_tc, table)
    return out.reshape(B, TOK, D)


# speedup vs baseline: 1.0832x; 1.0832x over previous
import functools

import jax
import jax.numpy as jnp
from jax import lax
from jax.experimental import pallas as pl
from jax.experimental.pallas import tpu as pltpu
from jax.experimental.pallas import tpu_sc as plsc

L_TX, L_SP, L_TP = 16, 8, 8
D = 768
B = 4096
TOK = L_TX + L_SP + L_TP
ROWS = B * TOK

NC, NS = 2, 16
NW = NC * NS
SC_ROWS = 65536
TC_ROWS = ROWS - SC_ROWS
ROWS_PER_W = SC_ROWS // NW
CHUNK = 64
NCHUNK = ROWS_PER_W // CHUNK
LANES = 16
DSTEPS = D // LANES
TC_BLK = 2048
TC_NB = TC_ROWS // TC_BLK


@functools.partial(
    pl.kernel,
    out_type=jax.ShapeDtypeStruct((ROWS * D,), jnp.float32),
    mesh=plsc.VectorSubcoreMesh(core_axis_name="c", subcore_axis_name="s"),
    scratch_types=[
        pltpu.VMEM((ROWS_PER_W,), jnp.int32),
        pltpu.VMEM((TOK * D,), jnp.float32),
        pltpu.VMEM((CHUNK * D,), jnp.float32),
        pltpu.VMEM((CHUNK * D,), jnp.float32),
        pltpu.SemaphoreType.DMA,
        pltpu.SemaphoreType.DMA,
    ],
)
def _sc_gather(table_hbm, idx_hbm, out_hbm, idx_v, tab_v, buf0, buf1,
               ss0, ss1):
    wid = lax.axis_index("s") * NC + lax.axis_index("c")
    base = wid * ROWS_PER_W
    pltpu.sync_copy(table_hbm, tab_v)
    pltpu.sync_copy(idx_hbm.at[pl.ds(base, ROWS_PER_W)], idx_v)

    bufs = (buf0, buf1)
    ssem = (ss0, ss1)

    LAG = 8

    def build(b, c):
        def group(g, carry):
            srcs = idx_v[pl.ds(c * CHUNK + g * LANES, LANES)] * D
            dst0 = g * LANES * D
            for j in range(LANES):
                src = srcs[j]
                dst = dst0 + j * D
                vals = [None] * DSTEPS
                for d in range(DSTEPS):
                    vals[d] = tab_v[pl.ds(src + d * LANES, LANES)]
                    if d >= LAG:
                        bufs[b][pl.ds(dst + (d - LAG) * LANES, LANES)] = (
                            vals[d - LAG])
                for d in range(DSTEPS - LAG, DSTEPS):
                    bufs[b][pl.ds(dst + d * LANES, LANES)] = vals[d]
            return carry
        lax.fori_loop(0, CHUNK // LANES, group, 0)

    def start_scatter(b, c):
        pltpu.async_copy(
            bufs[b], out_hbm.at[pl.ds((base + c * CHUNK) * D, CHUNK * D)],
            ssem[b])

    def wait_scatter(b):
        pltpu.make_async_copy(
            bufs[b], out_hbm.at[pl.ds(base * D, CHUNK * D)], ssem[b]).wait()

    def body(i, carry):
        c0 = i * 2

        @pl.when(i > 0)
        def _():
            wait_scatter(0)
        build(0, c0)
        start_scatter(0, c0)

        @pl.when(i > 0)
        def _():
            wait_scatter(1)
        build(1, c0 + 1)
        start_scatter(1, c0 + 1)
        return carry

    lax.fori_loop(0, NCHUNK // 2, body, 0)
    wait_scatter(0)
    wait_scatter(1)


def _tc_body(full_ref, idx_ref, tab_ref, out_ref):
    idxv = idx_ref[0, 0, :]
    onehot = (idxv[:, None]
              == lax.broadcasted_iota(jnp.int32, (TC_BLK, TOK), 1))
    out_ref[...] = jnp.dot(onehot.astype(jnp.float32), tab_ref[...],
                           preferred_element_type=jnp.float32)


_tc_fill = pl.pallas_call(
    _tc_body,
    grid=(TC_NB,),
    in_specs=[
        pl.BlockSpec(memory_space=pl.ANY),
        pl.BlockSpec((1, 1, TC_BLK), lambda i: (i, 0, 0)),
        pl.BlockSpec((TOK, D), lambda i: (0, 0)),
    ],
    out_specs=pl.BlockSpec((TC_BLK, D), lambda i: (SC_ROWS // TC_BLK + i, 0)),
    out_shape=jax.ShapeDtypeStruct((ROWS, D), jnp.float32),
    input_output_aliases={0: 0},
    compiler_params=pltpu.CompilerParams(
        dimension_semantics=("arbitrary",)),
)


def kernel(P_gn_txt, P_gn_ViT, P_gn_temp, idx_txt, idx_vit, idx_temp):
    table = jnp.concatenate([P_gn_txt, P_gn_ViT, P_gn_temp], axis=0)
    idx = jnp.concatenate(
        [idx_txt, idx_vit + L_TX, idx_temp + (L_TX + L_SP)], axis=1
    ).reshape(ROWS)
    sc_part = _sc_gather(table.reshape(TOK * D), idx)
    idx_tc = idx[SC_ROWS:].reshape(TC_NB, 1, TC_BLK)
    out = _tc_fill(sc_part.reshape(ROWS, D), idx_tc, table)
    return out.reshape(B, TOK, D)
